# baseline (speedup 1.0000x reference)
import jax
import jax.numpy as jnp
from jax import lax
from jax.experimental import pallas as pl
from jax.experimental.pallas import tpu as pltpu
from jax.experimental.pallas import tpu_sc as plsc

B = 16384
F_CAT = 26
VOCAB = 100000
F_NUM = 13
D = 32
F_OUT = F_CAT + F_NUM

NC = 2
NS = 16
NW = NC * NS
L = 16

BC = 16
ROWS_PER_W = B // NW
CHUNKS = ROWS_PER_W // BC
G = 4 * F_CAT
NG = (BC * F_CAT) // G
NT = (BC * F_NUM) // L


def _rsqrt_vec(x):
    i = plsc.bitcast(x, jnp.int32)
    y = plsc.bitcast(jnp.int32(0x5F3759DF) - (i >> 1), jnp.float32)
    xh = x * 0.5
    for _ in range(3):
        y = y * (1.5 - xh * y * y)
    return y


def _body(cat_hbm, num_hbm, tab_hbm, w_hbm, b_hbm, g_hbm, bt_hbm, out_hbm,
          cat_v, idx_v, num_v, catb_v, numb_v, par_v, gsem, osem):
    wid = lax.axis_index("s") * NC + lax.axis_index("c")
    base = wid * ROWS_PER_W

    pltpu.sync_copy(w_hbm, par_v.at[0])
    pltpu.sync_copy(b_hbm, par_v.at[1])
    pltpu.sync_copy(g_hbm, par_v.at[2])
    pltpu.sync_copy(bt_hbm, par_v.at[3])
    w0 = par_v[0, pl.ds(0, L)]
    w1 = par_v[0, pl.ds(L, L)]
    bb0 = par_v[1, pl.ds(0, L)]
    bb1 = par_v[1, pl.ds(L, L)]
    g0 = par_v[2, pl.ds(0, L)]
    g1 = par_v[2, pl.ds(L, L)]
    bt0 = par_v[3, pl.ds(0, L)]
    bt1 = par_v[3, pl.ds(L, L)]

    def _lanesum(v):
        s = v[0]
        for i in range(1, L):
            s = s + v[i]
        return s

    sw = _lanesum(w0 + w1)
    sb = _lanesum(bb0 + bb1)
    sww = _lanesum(w0 * w0 + w1 * w1)
    swb = _lanesum(w0 * bb0 + w1 * bb1)
    sbb = _lanesum(bb0 * bb0 + bb1 * bb1)
    inv_d = jnp.float32(1.0 / D)
    mw = sw * inv_d
    mb = sb * inv_d
    a_m = sww * inv_d - mw * mw
    c_m = swb * inv_d - mw * mb
    v_m = sbb * inv_d - mb * mb
    wc0 = w0 - mw
    wc1 = w1 - mw
    bc0 = bb0 - mb
    bc1 = bb1 - mb
    c2 = c_m * 2.0
    veps = v_m + 1e-5
    a0 = wc0 * g0
    a1 = wc1 * g1
    cc0 = bc0 * g0
    cc1 = bc1 * g1

    iota = lax.iota(jnp.int32, L)
    offs = []
    for t in range(7):
        f16 = lax.rem(iota + (16 * t), jnp.int32(F_CAT))
        offs.append(f16 * VOCAB)

    def chunk_body(c, _):
        b0 = base + c * BC

        pltpu.sync_copy(cat_hbm.at[pl.ds(b0 * F_CAT, BC * F_CAT)], cat_v)
        pltpu.sync_copy(num_hbm.at[pl.ds(b0 * F_NUM, BC * F_NUM)], num_v)

        for q in range(NG):
            for t in range(7):
                src = plsc.load_gather(cat_v, [iota + (G * q + 16 * t)])
                idx_v[q, pl.ds(16 * t, L)] = src + offs[t]

        gathers = []
        for q in range(NG):
            gathers.append(
                pltpu.async_copy(
                    tab_hbm.at[idx_v.at[q, pl.ds(0, G)]],
                    catb_v.at[pl.ds(G * q, G)],
                    gsem,
                )
            )

        for t in range(NT):
            x = num_v[pl.ds(t * L, L)]
            var = x * x * a_m + x * c2 + veps
            r = _rsqrt_vec(var)
            xr = x * r
            for l in range(L):
                p = t * L + l
                xs = xr[l]
                rs = r[l]
                numb_v[p, pl.ds(0, L)] = xs * a0 + (rs * cc0 + bt0)
                numb_v[p, pl.ds(L, L)] = xs * a1 + (rs * cc1 + bt1)

        writes = []
        for b in range(BC):
            writes.append(
                pltpu.async_copy(
                    numb_v.at[pl.ds(F_NUM * b, F_NUM)],
                    out_hbm.at[b0 + b, pl.ds(F_CAT, F_NUM)],
                    osem,
                )
            )

        for cp in gathers:
            cp.wait()

        for b in range(BC):
            writes.append(
                pltpu.async_copy(
                    catb_v.at[pl.ds(F_CAT * b, F_CAT)],
                    out_hbm.at[b0 + b, pl.ds(0, F_CAT)],
                    osem,
                )
            )

        for cp in writes:
            cp.wait()
        return 0

    lax.fori_loop(0, CHUNKS, chunk_body, 0)


@jax.jit
def _run(cat_flat, num_flat, tab_flat, w, b, g, bt):
    mesh = plsc.VectorSubcoreMesh(
        core_axis_name="c", subcore_axis_name="s", num_cores=NC, num_subcores=NS
    )
    return pl.kernel(
        _body,
        out_type=jax.ShapeDtypeStruct((B, F_OUT, D), jnp.float32),
        mesh=mesh,
        compiler_params=pltpu.CompilerParams(
            needs_layout_passes=False, use_tc_tiling_on_sc=False),
        scratch_types=[
            pltpu.VMEM((BC * F_CAT,), jnp.int32),
            pltpu.VMEM((NG, 128), jnp.int32),
            pltpu.VMEM((BC * F_NUM,), jnp.float32),
            pltpu.VMEM((BC * F_CAT, D), jnp.float32),
            pltpu.VMEM((BC * F_NUM, D), jnp.float32),
            pltpu.VMEM((4, D), jnp.float32),
            pltpu.SemaphoreType.DMA,
            pltpu.SemaphoreType.DMA,
        ],
    )(cat_flat, num_flat, tab_flat, w, b, g, bt)


def kernel(categorical_features, numerical_features, tables, W_num, b_num,
           ln_gamma, ln_beta):
    cat_flat = categorical_features.astype(jnp.int32).reshape(-1)
    num_flat = numerical_features.reshape(-1)
    tab_flat = tables.reshape(F_CAT * VOCAB, D)
    return _run(cat_flat, num_flat, tab_flat, W_num, b_num, ln_gamma, ln_beta)
